# baseline (speedup 1.0000x reference)
import jax
import jax.numpy as jnp
from jax import lax
from jax.experimental import pallas as pl
from jax.experimental.pallas import tpu as pltpu
from jax.experimental.pallas import tpu_sc as plsc

NC = 2
NS = 16
LN = 16
CHUNK = 128
NPAD = 10240
RB = 1000


def _seg_sum_sc(table, src3, dst3, split_feat, with_counts, grp_chunks):
    dh = table.shape[1]
    nchunks = src3.shape[1]
    G = grp_chunks
    ngroups = nchunks // G
    rows_per_tile = NPAD // NS
    nread = rows_per_tile // CHUNK
    nlr = dh // LN

    mesh = plsc.VectorSubcoreMesh(
        core_axis_name="c", subcore_axis_name="s",
        num_cores=NC, num_subcores=NS)

    outs = [jax.ShapeDtypeStruct((NC, NPAD, dh), jnp.float32)]
    if with_counts:
        outs.append(jax.ShapeDtypeStruct((NC, NPAD, LN), jnp.float32))

    scratch = [
        pltpu.VMEM((CHUNK,), jnp.int32),
        pltpu.VMEM((CHUNK,), jnp.int32),
        pltpu.VMEM((CHUNK, dh), jnp.float32),
        pltpu.VMEM((CHUNK, LN), jnp.float32),
        pltpu.VMEM_SHARED((NPAD, dh), jnp.float32),
        pltpu.SemaphoreType.DMA,
    ]
    if with_counts:
        scratch.insert(5, pltpu.VMEM_SHARED((NPAD, LN), jnp.float32))

    def body(x_hbm, src_hbm, dst_hbm, *refs):
        if with_counts:
            (sum_hbm, cnt_hbm, src_v, dst_v, rows_v, ones_v,
             acc, cacc, sem) = refs
        else:
            (sum_hbm, src_v, dst_v, rows_v, ones_v,
             acc, sem) = refs
        c = lax.axis_index("c")
        s = lax.axis_index("s")
        t = s if split_feat else c * NS + s

        zbuf = rows_v

        def zr(r, _):
            for k in range(nlr):
                zbuf[r, pl.ds(k * LN, LN)] = jnp.zeros((LN,), jnp.float32)
            return 0
        lax.fori_loop(0, CHUNK, zr, 0)

        def zo(i, _):
            ones_v[i, pl.ds(0, LN)] = jnp.zeros((LN,), jnp.float32)
            return 0
        lax.fori_loop(0, CHUNK, zo, 0)

        def zs(j, _):
            r0 = s * rows_per_tile + j * CHUNK
            pltpu.sync_copy(zbuf, acc.at[pl.ds(r0, CHUNK)])
            if with_counts:
                pltpu.sync_copy(ones_v, cacc.at[pl.ds(r0, CHUNK)])
            return 0
        lax.fori_loop(0, nread, zs, 0)

        if with_counts:
            def fo(i, _):
                ones_v[i, pl.ds(0, LN)] = jnp.ones((LN,), jnp.float32)
                return 0
            lax.fori_loop(0, CHUNK, fo, 0)

        plsc.subcore_barrier()

        def ch(j, _):
            pltpu.sync_copy(src_hbm.at[t, j], src_v)
            pltpu.sync_copy(dst_hbm.at[t, j], dst_v)
            if split_feat:
                for k in range(CHUNK // LN):
                    src_v[pl.ds(k * LN, LN)] = \
                        src_v[pl.ds(k * LN, LN)] * 2 + c
            pltpu.async_copy(x_hbm.at[src_v], rows_v, sem).wait()
            pltpu.sync_copy(rows_v, acc.at[dst_v], add=True)
            if with_counts:
                pltpu.sync_copy(ones_v, cacc.at[dst_v], add=True)
            return 0
        lax.fori_loop(0, nchunks, ch, 0)

        plsc.subcore_barrier()

        def rd(j, _):
            r0 = s * rows_per_tile + j * CHUNK
            pltpu.sync_copy(acc.at[pl.ds(r0, CHUNK)], zbuf)
            pltpu.sync_copy(zbuf, sum_hbm.at[c, pl.ds(r0, CHUNK)])
            if with_counts:
                pltpu.sync_copy(cacc.at[pl.ds(r0, CHUNK)], ones_v)
                pltpu.sync_copy(ones_v, cnt_hbm.at[c, pl.ds(r0, CHUNK)])
            return 0
        lax.fori_loop(0, nread, rd, 0)

    f = pl.kernel(body, out_type=tuple(outs), mesh=mesh,
                  scratch_types=tuple(scratch),
                  compiler_params=pltpu.CompilerParams(
                      use_tc_tiling_on_sc=False))
    return f(table, src3, dst3)


def _dot_nt(a, b):
    return lax.dot_general(a, b, (((1,), (1,)), ((), ())),
                           preferred_element_type=jnp.float32)


def _mean_block(sums_ref, cnt_ref, split_feat):
    inv = 1.0 / jnp.maximum(cnt_ref[0, :, 0:1] + cnt_ref[1, :, 0:1], 1.0)
    if split_feat:
        summed = jnp.concatenate([sums_ref[0], sums_ref[1]], axis=1)
    else:
        summed = sums_ref[0] + sums_ref[1]
    return summed * inv


def _dense_layer(sums, cnt, x, wl, bl, wr, g, b, n, split_feat):
    nrb = n // RB
    dh = sums.shape[2]
    dout = wl.shape[0]
    din = x.shape[1]

    def body(sums_ref, cnt_ref, x_ref, wl_ref, bl_ref, wr_ref, g_ref, b_ref,
             out_ref, t_s, st_s):
        p = pl.program_id(0)
        i = pl.program_id(1)

        @pl.when(p == 0)
        def _():
            mean = _mean_block(sums_ref, cnt_ref, split_feat)
            t = _dot_nt(mean, wl_ref[...]) + bl_ref[...] + \
                _dot_nt(x_ref[...], wr_ref[...])
            t_s[pl.ds(i * RB, RB), :] = t

            @pl.when(i == 0)
            def _():
                st_s[...] = jnp.zeros((2, dout), jnp.float32)
            st_s[0:1, :] += jnp.sum(t, axis=0, keepdims=True)
            st_s[1:2, :] += jnp.sum(t * t, axis=0, keepdims=True)

        @pl.when(p == 1)
        def _():
            t = t_s[pl.ds(i * RB, RB), :]
            mu = st_s[0:1, :] * (1.0 / n)
            var = st_s[1:2, :] * (1.0 / n) - mu * mu
            y = (t - mu) * lax.rsqrt(var + 1e-5) * g_ref[...] + b_ref[...]
            out_ref[...] = jnp.maximum(y, 0.0)

    return pl.pallas_call(
        body,
        grid=(2, nrb),
        in_specs=[
            pl.BlockSpec((NC, RB, dh), lambda p, i: (0, i, 0)),
            pl.BlockSpec((NC, RB, LN), lambda p, i: (0, i, 0)),
            pl.BlockSpec((RB, din), lambda p, i: (i, 0)),
            pl.BlockSpec((dout, din), lambda p, i: (0, 0)),
            pl.BlockSpec((1, dout), lambda p, i: (0, 0)),
            pl.BlockSpec((dout, din), lambda p, i: (0, 0)),
            pl.BlockSpec((1, dout), lambda p, i: (0, 0)),
            pl.BlockSpec((1, dout), lambda p, i: (0, 0)),
        ],
        out_specs=pl.BlockSpec((RB, dout), lambda p, i: (i, 0)),
        out_shape=jax.ShapeDtypeStruct((n, dout), jnp.float32),
        scratch_shapes=[
            pltpu.VMEM((n, dout), jnp.float32),
            pltpu.VMEM((2, dout), jnp.float32),
        ],
    )(sums, cnt, x, wl, bl, wr, g, b)


def _dense_layer_pool(sums, cnt, x, batch2, wl, bl, wr, g, b, n, ng,
                      split_feat):
    nrb = n // RB
    dh = sums.shape[2]
    dout = wl.shape[0]
    din = x.shape[1]

    def body(sums_ref, cnt_ref, x_ref, batch_ref, wl_ref, bl_ref, wr_ref,
             g_ref, b_ref, out_ref, t_s, st_s, ps_s, pm_s, pc_s):
        p = pl.program_id(0)
        i = pl.program_id(1)

        @pl.when(p == 0)
        def _():
            mean = _mean_block(sums_ref, cnt_ref, split_feat)
            t = _dot_nt(mean, wl_ref[...]) + bl_ref[...] + \
                _dot_nt(x_ref[...], wr_ref[...])
            t_s[pl.ds(i * RB, RB), :] = t

            @pl.when(i == 0)
            def _():
                st_s[...] = jnp.zeros((2, dout), jnp.float32)
            st_s[0:1, :] += jnp.sum(t, axis=0, keepdims=True)
            st_s[1:2, :] += jnp.sum(t * t, axis=0, keepdims=True)

        @pl.when(p == 1)
        def _():
            t = t_s[pl.ds(i * RB, RB), :]
            mu = st_s[0:1, :] * (1.0 / n)
            var = st_s[1:2, :] * (1.0 / n) - mu * mu
            f = jnp.maximum(
                (t - mu) * lax.rsqrt(var + 1e-5) * g_ref[...] + b_ref[...],
                0.0)

            @pl.when(i == 0)
            def _():
                ps_s[...] = jnp.zeros((ng, dout), jnp.float32)
                pm_s[...] = jnp.full((ng, dout), -jnp.inf, jnp.float32)
                pc_s[...] = jnp.zeros((ng, 1), jnp.float32)

            bb = batch_ref[...]
            onehot = (bb == lax.broadcasted_iota(jnp.int32, (RB, ng), 1)
                      ).astype(jnp.float32)
            ps_s[...] += lax.dot_general(
                onehot, f, (((0,), (0,)), ((), ())),
                preferred_element_type=jnp.float32)
            pc_s[...] += lax.dot_general(
                onehot, jnp.ones((RB, 1), jnp.float32),
                (((0,), (0,)), ((), ())), preferred_element_type=jnp.float32)

            lo = batch_ref[0, 0]
            hi = batch_ref[RB - 1, 0]

            def gmax(gi, _):
                mk = bb == gi
                fm = jnp.max(jnp.where(mk, f, -jnp.inf), axis=0,
                             keepdims=True)
                pm_s[pl.ds(gi, 1), :] = jnp.maximum(pm_s[pl.ds(gi, 1), :], fm)
                return 0
            lax.fori_loop(lo, hi + 1, gmax, 0)

            @pl.when(i == nrb - 1)
            def _():
                cg = jnp.maximum(pc_s[...], 1.0)
                out_ref[...] = jnp.concatenate(
                    [ps_s[...] / cg, pm_s[...]], axis=1)

    return pl.pallas_call(
        body,
        grid=(2, nrb),
        in_specs=[
            pl.BlockSpec((NC, RB, dh), lambda p, i: (0, i, 0)),
            pl.BlockSpec((NC, RB, LN), lambda p, i: (0, i, 0)),
            pl.BlockSpec((RB, din), lambda p, i: (i, 0)),
            pl.BlockSpec((RB, 1), lambda p, i: (i, 0)),
            pl.BlockSpec((dout, din), lambda p, i: (0, 0)),
            pl.BlockSpec((1, dout), lambda p, i: (0, 0)),
            pl.BlockSpec((dout, din), lambda p, i: (0, 0)),
            pl.BlockSpec((1, dout), lambda p, i: (0, 0)),
            pl.BlockSpec((1, dout), lambda p, i: (0, 0)),
        ],
        out_specs=pl.BlockSpec((ng, 2 * dout), lambda p, i: (0, 0)),
        out_shape=jax.ShapeDtypeStruct((ng, 2 * dout), jnp.float32),
        scratch_shapes=[
            pltpu.VMEM((n, dout), jnp.float32),
            pltpu.VMEM((2, dout), jnp.float32),
            pltpu.VMEM((ng, dout), jnp.float32),
            pltpu.VMEM((ng, dout), jnp.float32),
            pltpu.VMEM((ng, 1), jnp.float32),
        ],
    )(sums, cnt, x, batch2, wl, bl, wr, g, b)


def _head(f0, f1, le0, le1, wf, bf, wc1, bc1, wc2, bc2):
    ng, dg = f0.shape
    nl = le0.shape[1]

    def body(f0_ref, f1_ref, le0_ref, le1_ref, wf_ref, bf_ref, wc1_ref,
             bc1_ref, wc2_ref, bc2_ref, out_ref):
        wf_main = wf_ref[:, 0:dg]
        wf_log = wf_ref[:, dg:dg + nl]
        fused0 = _dot_nt(f0_ref[...], wf_main) + \
            _dot_nt(le0_ref[...], wf_log) + bf_ref[...]
        fused1 = _dot_nt(f1_ref[...], wf_main) + \
            _dot_nt(le1_ref[...], wf_log) + bf_ref[...]
        comb = jnp.concatenate(
            [fused0, fused1, jnp.abs(fused0 - fused1), fused0 * fused1],
            axis=1)
        h = jnp.maximum(_dot_nt(comb, wc1_ref[...]) + bc1_ref[...], 0.0)
        z = _dot_nt(h, wc2_ref[...])[:, 0:1] + bc2_ref[0, 0]
        out_ref[...] = 1.0 / (1.0 + jnp.exp(-z))

    return pl.pallas_call(
        body,
        out_shape=jax.ShapeDtypeStruct((ng, 1), jnp.float32),
    )(f0, f1, le0, le1, wf, bf, wc1, bc1, wc2, bc2)


def kernel(x0, edge_index0, batch0, logic0_embed, x1, edge_index1, batch1,
           logic1_embed, Wl1, bl1, Wr1, Wl2, bl2, Wr2, g1, b1, g2, b2,
           Wf, bf, Wc1, bc1, Wc2, bc2):
    n, din = x0.shape
    dh = Wl1.shape[0]
    ng = logic0_embed.shape[0]

    bl1r = bl1.reshape(1, -1)
    g1r = g1.reshape(1, -1)
    b1r = b1.reshape(1, -1)
    bl2r = bl2.reshape(1, -1)
    g2r = g2.reshape(1, -1)
    b2r = b2.reshape(1, -1)
    bfr = bf.reshape(1, -1)
    bc1r = bc1.reshape(1, -1)
    bc2r = bc2.reshape(1, -1)
    wc2r = jnp.tile(Wc2, (8, 1))

    def side(x, ei, batch):
        e = ei.shape[1]
        unit = NC * NS * CHUNK * 16
        ep = -(-e // unit) * unit
        src = ei[0]
        dst = ei[1]
        if ep > e:
            pad = ep - e
            src = jnp.concatenate([src, jnp.zeros((pad,), jnp.int32)])
            dst = jnp.concatenate(
                [dst, jnp.full((pad,), NPAD - 8, jnp.int32)])
        src_e = src.reshape(NC * NS, -1, CHUNK)
        dst_e = dst.reshape(NC * NS, -1, CHUNK)
        src_f = src.reshape(NS, -1, CHUNK)
        dst_f = dst.reshape(NS, -1, CHUNK)
        sums1, cnt = _seg_sum_sc(x, src_e, dst_e, False, True, 8)
        h1 = _dense_layer(sums1, cnt, x, Wl1, bl1r, Wr1, g1r, b1r, n, False)
        sums2 = _seg_sum_sc(
            h1.reshape(2 * n, dh // 2), src_f, dst_f, True, False, 16)[0]
        pooled = _dense_layer_pool(
            sums2, cnt, h1, batch.reshape(n, 1), Wl2, bl2r, Wr2, g2r, b2r,
            n, ng, True)
        return pooled

    f0 = side(x0, edge_index0, batch0)
    f1 = side(x1, edge_index1, batch1)
    prob = _head(f0, f1, logic0_embed, logic1_embed, Wf, bfr, Wc1, bc1r,
                 wc2r, bc2r)
    return prob[:, 0]


# speedup vs baseline: 1.0008x; 1.0008x over previous
import jax
import jax.numpy as jnp
from jax import lax
from jax.experimental import pallas as pl
from jax.experimental.pallas import tpu as pltpu
from jax.experimental.pallas import tpu_sc as plsc

NC = 2
NS = 16
LN = 16
CHUNK = 128
NPAD = 10240
RB = 1000


def _seg_sum_sc(table, src3, dst3, split_feat, with_counts):
    dh = table.shape[1]
    ep = src3.shape[0]
    ept = ep // NS if split_feat else ep // (NC * NS)
    nchunks = ept // CHUNK
    rows_per_tile = NPAD // NS
    nread = rows_per_tile // CHUNK
    nlr = dh // LN

    mesh = plsc.VectorSubcoreMesh(
        core_axis_name="c", subcore_axis_name="s",
        num_cores=NC, num_subcores=NS)

    outs = [jax.ShapeDtypeStruct((NC, NPAD, dh), jnp.float32)]
    if with_counts:
        outs.append(jax.ShapeDtypeStruct((NC, NPAD, LN), jnp.float32))

    scratch = [
        pltpu.VMEM((CHUNK,), jnp.int32),
        pltpu.VMEM((CHUNK,), jnp.int32),
        pltpu.VMEM((CHUNK, dh), jnp.float32),
        pltpu.VMEM((CHUNK, LN), jnp.float32),
        pltpu.VMEM_SHARED((NPAD, dh), jnp.float32),
        pltpu.SemaphoreType.DMA,
    ]
    if with_counts:
        scratch.insert(5, pltpu.VMEM_SHARED((NPAD, LN), jnp.float32))

    def body(x_hbm, src_hbm, dst_hbm, *refs):
        if with_counts:
            (sum_hbm, cnt_hbm, src_v, dst_v, rows_v, ones_v,
             acc, cacc, sem) = refs
        else:
            (sum_hbm, src_v, dst_v, rows_v, ones_v,
             acc, sem) = refs
        c = lax.axis_index("c")
        s = lax.axis_index("s")
        t = s if split_feat else c * NS + s

        zbuf = rows_v

        def zr(r, _):
            for k in range(nlr):
                zbuf[r, pl.ds(k * LN, LN)] = jnp.zeros((LN,), jnp.float32)
            return 0
        lax.fori_loop(0, CHUNK, zr, 0)

        def zo(i, _):
            ones_v[i, pl.ds(0, LN)] = jnp.zeros((LN,), jnp.float32)
            return 0
        lax.fori_loop(0, CHUNK, zo, 0)

        def zs(j, _):
            r0 = s * rows_per_tile + j * CHUNK
            pltpu.sync_copy(zbuf, acc.at[pl.ds(r0, CHUNK)])
            if with_counts:
                pltpu.sync_copy(ones_v, cacc.at[pl.ds(r0, CHUNK)])
            return 0
        lax.fori_loop(0, nread, zs, 0)

        if with_counts:
            def fo(i, _):
                ones_v[i, pl.ds(0, LN)] = jnp.ones((LN,), jnp.float32)
                return 0
            lax.fori_loop(0, CHUNK, fo, 0)

        plsc.subcore_barrier()

        def ch(j, _):
            base = t * ept + j * CHUNK
            pltpu.sync_copy(src_hbm.at[pl.ds(base, CHUNK)], src_v)
            pltpu.sync_copy(dst_hbm.at[pl.ds(base, CHUNK)], dst_v)
            if split_feat:
                for k in range(CHUNK // LN):
                    src_v[pl.ds(k * LN, LN)] = \
                        src_v[pl.ds(k * LN, LN)] * 2 + c
            pltpu.async_copy(x_hbm.at[src_v], rows_v, sem).wait()
            pltpu.sync_copy(rows_v, acc.at[dst_v], add=True)
            if with_counts:
                pltpu.sync_copy(ones_v, cacc.at[dst_v], add=True)
            return 0
        lax.fori_loop(0, nchunks, ch, 0)

        plsc.subcore_barrier()

        def rd(j, _):
            r0 = s * rows_per_tile + j * CHUNK
            pltpu.sync_copy(acc.at[pl.ds(r0, CHUNK)], zbuf)
            pltpu.sync_copy(zbuf, sum_hbm.at[c, pl.ds(r0, CHUNK)])
            if with_counts:
                pltpu.sync_copy(cacc.at[pl.ds(r0, CHUNK)], ones_v)
                pltpu.sync_copy(ones_v, cnt_hbm.at[c, pl.ds(r0, CHUNK)])
            return 0
        lax.fori_loop(0, nread, rd, 0)

    f = pl.kernel(body, out_type=tuple(outs), mesh=mesh,
                  scratch_types=tuple(scratch),
                  compiler_params=pltpu.CompilerParams(
                      use_tc_tiling_on_sc=False))
    return f(table, src3, dst3)


def _dot_nt(a, b):
    return lax.dot_general(a, b, (((1,), (1,)), ((), ())),
                           preferred_element_type=jnp.float32)


def _mean_block(sums_ref, cnt_ref, split_feat):
    inv = 1.0 / jnp.maximum(cnt_ref[0, :, 0:1] + cnt_ref[1, :, 0:1], 1.0)
    if split_feat:
        summed = jnp.concatenate([sums_ref[0], sums_ref[1]], axis=1)
    else:
        summed = sums_ref[0] + sums_ref[1]
    return summed * inv


def _dense_layer(sums, cnt, x, wl, bl, wr, g, b, n, split_feat):
    nrb = n // RB
    dh = sums.shape[2]
    dout = wl.shape[0]
    din = x.shape[1]

    def body(sums_ref, cnt_ref, x_ref, wl_ref, bl_ref, wr_ref, g_ref, b_ref,
             out_ref, t_s, st_s):
        p = pl.program_id(0)
        i = pl.program_id(1)

        @pl.when(p == 0)
        def _():
            mean = _mean_block(sums_ref, cnt_ref, split_feat)
            t = _dot_nt(mean, wl_ref[...]) + bl_ref[...] + \
                _dot_nt(x_ref[...], wr_ref[...])
            t_s[pl.ds(i * RB, RB), :] = t

            @pl.when(i == 0)
            def _():
                st_s[...] = jnp.zeros((2, dout), jnp.float32)
            st_s[0:1, :] += jnp.sum(t, axis=0, keepdims=True)
            st_s[1:2, :] += jnp.sum(t * t, axis=0, keepdims=True)

        @pl.when(p == 1)
        def _():
            t = t_s[pl.ds(i * RB, RB), :]
            mu = st_s[0:1, :] * (1.0 / n)
            var = st_s[1:2, :] * (1.0 / n) - mu * mu
            y = (t - mu) * lax.rsqrt(var + 1e-5) * g_ref[...] + b_ref[...]
            out_ref[...] = jnp.maximum(y, 0.0)

    return pl.pallas_call(
        body,
        grid=(2, nrb),
        in_specs=[
            pl.BlockSpec((NC, RB, dh), lambda p, i: (0, i, 0)),
            pl.BlockSpec((NC, RB, LN), lambda p, i: (0, i, 0)),
            pl.BlockSpec((RB, din), lambda p, i: (i, 0)),
            pl.BlockSpec((dout, din), lambda p, i: (0, 0)),
            pl.BlockSpec((1, dout), lambda p, i: (0, 0)),
            pl.BlockSpec((dout, din), lambda p, i: (0, 0)),
            pl.BlockSpec((1, dout), lambda p, i: (0, 0)),
            pl.BlockSpec((1, dout), lambda p, i: (0, 0)),
        ],
        out_specs=pl.BlockSpec((RB, dout), lambda p, i: (i, 0)),
        out_shape=jax.ShapeDtypeStruct((n, dout), jnp.float32),
        scratch_shapes=[
            pltpu.VMEM((n, dout), jnp.float32),
            pltpu.VMEM((2, dout), jnp.float32),
        ],
    )(sums, cnt, x, wl, bl, wr, g, b)


def _dense_layer_pool(sums, cnt, x, batch2, wl, bl, wr, g, b, n, ng,
                      split_feat):
    nrb = n // RB
    dh = sums.shape[2]
    dout = wl.shape[0]
    din = x.shape[1]

    def body(sums_ref, cnt_ref, x_ref, batch_ref, wl_ref, bl_ref, wr_ref,
             g_ref, b_ref, out_ref, t_s, st_s, ps_s, pm_s, pc_s):
        p = pl.program_id(0)
        i = pl.program_id(1)

        @pl.when(p == 0)
        def _():
            mean = _mean_block(sums_ref, cnt_ref, split_feat)
            t = _dot_nt(mean, wl_ref[...]) + bl_ref[...] + \
                _dot_nt(x_ref[...], wr_ref[...])
            t_s[pl.ds(i * RB, RB), :] = t

            @pl.when(i == 0)
            def _():
                st_s[...] = jnp.zeros((2, dout), jnp.float32)
            st_s[0:1, :] += jnp.sum(t, axis=0, keepdims=True)
            st_s[1:2, :] += jnp.sum(t * t, axis=0, keepdims=True)

        @pl.when(p == 1)
        def _():
            t = t_s[pl.ds(i * RB, RB), :]
            mu = st_s[0:1, :] * (1.0 / n)
            var = st_s[1:2, :] * (1.0 / n) - mu * mu
            f = jnp.maximum(
                (t - mu) * lax.rsqrt(var + 1e-5) * g_ref[...] + b_ref[...],
                0.0)

            @pl.when(i == 0)
            def _():
                ps_s[...] = jnp.zeros((ng, dout), jnp.float32)
                pm_s[...] = jnp.full((ng, dout), -jnp.inf, jnp.float32)
                pc_s[...] = jnp.zeros((ng, 1), jnp.float32)

            bb = batch_ref[...]
            lo = batch_ref[0, 0]
            hi = batch_ref[RB - 1, 0]

            def gpool(gi, _):
                mk = bb == gi
                fs = jnp.sum(jnp.where(mk, f, 0.0), axis=0, keepdims=True)
                fm = jnp.max(jnp.where(mk, f, -jnp.inf), axis=0,
                             keepdims=True)
                cs = jnp.sum(mk.astype(jnp.float32), axis=0, keepdims=True)
                ps_s[pl.ds(gi, 1), :] += fs
                pm_s[pl.ds(gi, 1), :] = jnp.maximum(pm_s[pl.ds(gi, 1), :], fm)
                pc_s[pl.ds(gi, 1), :] += cs
                return 0
            lax.fori_loop(lo, hi + 1, gpool, 0)

            @pl.when(i == nrb - 1)
            def _():
                cg = jnp.maximum(pc_s[...], 1.0)
                out_ref[...] = jnp.concatenate(
                    [ps_s[...] / cg, pm_s[...]], axis=1)

    return pl.pallas_call(
        body,
        grid=(2, nrb),
        in_specs=[
            pl.BlockSpec((NC, RB, dh), lambda p, i: (0, i, 0)),
            pl.BlockSpec((NC, RB, LN), lambda p, i: (0, i, 0)),
            pl.BlockSpec((RB, din), lambda p, i: (i, 0)),
            pl.BlockSpec((RB, 1), lambda p, i: (i, 0)),
            pl.BlockSpec((dout, din), lambda p, i: (0, 0)),
            pl.BlockSpec((1, dout), lambda p, i: (0, 0)),
            pl.BlockSpec((dout, din), lambda p, i: (0, 0)),
            pl.BlockSpec((1, dout), lambda p, i: (0, 0)),
            pl.BlockSpec((1, dout), lambda p, i: (0, 0)),
        ],
        out_specs=pl.BlockSpec((ng, 2 * dout), lambda p, i: (0, 0)),
        out_shape=jax.ShapeDtypeStruct((ng, 2 * dout), jnp.float32),
        scratch_shapes=[
            pltpu.VMEM((n, dout), jnp.float32),
            pltpu.VMEM((2, dout), jnp.float32),
            pltpu.VMEM((ng, dout), jnp.float32),
            pltpu.VMEM((ng, dout), jnp.float32),
            pltpu.VMEM((ng, 1), jnp.float32),
        ],
    )(sums, cnt, x, batch2, wl, bl, wr, g, b)


def _head(f0, f1, le0, le1, wf, bf, wc1, bc1, wc2, bc2):
    ng, dg = f0.shape
    nl = le0.shape[1]

    def body(f0_ref, f1_ref, le0_ref, le1_ref, wf_ref, bf_ref, wc1_ref,
             bc1_ref, wc2_ref, bc2_ref, out_ref):
        wf_main = wf_ref[:, 0:dg]
        wf_log = wf_ref[:, dg:dg + nl]
        fused0 = _dot_nt(f0_ref[...], wf_main) + \
            _dot_nt(le0_ref[...], wf_log) + bf_ref[...]
        fused1 = _dot_nt(f1_ref[...], wf_main) + \
            _dot_nt(le1_ref[...], wf_log) + bf_ref[...]
        comb = jnp.concatenate(
            [fused0, fused1, jnp.abs(fused0 - fused1), fused0 * fused1],
            axis=1)
        h = jnp.maximum(_dot_nt(comb, wc1_ref[...]) + bc1_ref[...], 0.0)
        z = _dot_nt(h, wc2_ref[...])[:, 0:1] + bc2_ref[0, 0]
        out_ref[...] = 1.0 / (1.0 + jnp.exp(-z))

    return pl.pallas_call(
        body,
        out_shape=jax.ShapeDtypeStruct((ng, 1), jnp.float32),
    )(f0, f1, le0, le1, wf, bf, wc1, bc1, wc2, bc2)


def kernel(x0, edge_index0, batch0, logic0_embed, x1, edge_index1, batch1,
           logic1_embed, Wl1, bl1, Wr1, Wl2, bl2, Wr2, g1, b1, g2, b2,
           Wf, bf, Wc1, bc1, Wc2, bc2):
    n, din = x0.shape
    dh = Wl1.shape[0]
    ng = logic0_embed.shape[0]

    bl1r = bl1.reshape(1, -1)
    g1r = g1.reshape(1, -1)
    b1r = b1.reshape(1, -1)
    bl2r = bl2.reshape(1, -1)
    g2r = g2.reshape(1, -1)
    b2r = b2.reshape(1, -1)
    bfr = bf.reshape(1, -1)
    bc1r = bc1.reshape(1, -1)
    bc2r = bc2.reshape(1, -1)
    wc2r = jnp.tile(Wc2, (8, 1))

    def side(x, ei, batch):
        e = ei.shape[1]
        unit = NC * NS * CHUNK * 16
        ep = -(-e // unit) * unit
        src = ei[0]
        dst = ei[1]
        if ep > e:
            pad = ep - e
            src = jnp.concatenate([src, jnp.zeros((pad,), jnp.int32)])
            dst = jnp.concatenate(
                [dst, jnp.full((pad,), NPAD - 8, jnp.int32)])
        sums1, cnt = _seg_sum_sc(x, src, dst, False, True)
        h1 = _dense_layer(sums1, cnt, x, Wl1, bl1r, Wr1, g1r, b1r, n, False)
        sums2 = _seg_sum_sc(
            h1.reshape(2 * n, dh // 2), src, dst, True, False)[0]
        pooled = _dense_layer_pool(
            sums2, cnt, h1, batch.reshape(n, 1), Wl2, bl2r, Wr2, g2r, b2r,
            n, ng, True)
        return pooled

    f0 = side(x0, edge_index0, batch0)
    f1 = side(x1, edge_index1, batch1)
    prob = _head(f0, f1, logic0_embed, logic1_embed, Wf, bfr, Wc1, bc1r,
                 wc2r, bc2r)
    return prob[:, 0]


# speedup vs baseline: 1.3414x; 1.3403x over previous
import jax
import jax.numpy as jnp
from jax import lax
from jax.experimental import pallas as pl
from jax.experimental.pallas import tpu as pltpu
from jax.experimental.pallas import tpu_sc as plsc

NC = 2
NS = 16
LN = 16
CHUNK = 128
NPAD = 10240
RB = 1000


def _seg_sum_sc(table, src3, dst3, split_feat, with_counts):
    dh = table.shape[1]
    ep = src3.shape[0]
    ept = ep // NS if split_feat else ep // (NC * NS)
    nchunks = ept // CHUNK
    rows_per_tile = NPAD // NS
    nread = rows_per_tile // CHUNK
    nlr = dh // LN

    mesh = plsc.VectorSubcoreMesh(
        core_axis_name="c", subcore_axis_name="s",
        num_cores=NC, num_subcores=NS)

    outs = [jax.ShapeDtypeStruct((NC, NPAD, dh), jnp.float32)]
    if with_counts:
        outs.append(jax.ShapeDtypeStruct((NC, NPAD, LN), jnp.float32))

    scratch = [
        pltpu.VMEM((CHUNK,), jnp.int32),
        pltpu.VMEM((CHUNK,), jnp.int32),
        pltpu.VMEM((CHUNK,), jnp.int32),
        pltpu.VMEM((CHUNK, dh), jnp.float32),
        pltpu.VMEM((CHUNK, LN), jnp.float32),
        pltpu.VMEM_SHARED((NPAD, dh), jnp.float32),
        pltpu.SemaphoreType.DMA,
    ]
    if with_counts:
        scratch.insert(6, pltpu.VMEM_SHARED((NPAD, LN), jnp.float32))

    def body(x_hbm, src_hbm, dst_hbm, *refs):
        if with_counts:
            (sum_hbm, cnt_hbm, src_v, idx_v, dst_v, rows_v, ones_v,
             acc, cacc, sem) = refs
        else:
            (sum_hbm, src_v, idx_v, dst_v, rows_v, ones_v,
             acc, sem) = refs
        c = lax.axis_index("c")
        s = lax.axis_index("s")
        t = s if split_feat else c * NS + s

        zbuf = rows_v

        def zr(r, _):
            for k in range(nlr):
                zbuf[r, pl.ds(k * LN, LN)] = jnp.zeros((LN,), jnp.float32)
            return 0
        lax.fori_loop(0, CHUNK, zr, 0)

        def zo(i, _):
            ones_v[i, pl.ds(0, LN)] = jnp.zeros((LN,), jnp.float32)
            return 0
        lax.fori_loop(0, CHUNK, zo, 0)

        def zs(j, _):
            r0 = s * rows_per_tile + j * CHUNK
            pltpu.sync_copy(zbuf, acc.at[pl.ds(r0, CHUNK)])
            if with_counts:
                pltpu.sync_copy(ones_v, cacc.at[pl.ds(r0, CHUNK)])
            return 0
        lax.fori_loop(0, nread, zs, 0)

        if with_counts:
            def fo(i, _):
                ones_v[i, pl.ds(0, LN)] = jnp.ones((LN,), jnp.float32)
                return 0
            lax.fori_loop(0, CHUNK, fo, 0)

        plsc.subcore_barrier()

        def ch(j, _):
            base = t * ept + j * CHUNK
            pltpu.sync_copy(src_hbm.at[pl.ds(base, CHUNK)], src_v)
            pltpu.sync_copy(dst_hbm.at[pl.ds(base, CHUNK)], dst_v)
            if split_feat:
                for k in range(CHUNK // LN):
                    idx_v[pl.ds(k * LN, LN)] = \
                        src_v[pl.ds(k * LN, LN)] * 2 + c
                gsrc = x_hbm.at[idx_v]
            else:
                gsrc = x_hbm.at[src_v]
            pltpu.async_copy(gsrc, rows_v, sem).wait()
            pltpu.sync_copy(rows_v, acc.at[dst_v], add=True)
            if with_counts:
                pltpu.sync_copy(ones_v, cacc.at[dst_v], add=True)
            return 0
        lax.fori_loop(0, nchunks, ch, 0)

        plsc.subcore_barrier()

        def rd(j, _):
            r0 = s * rows_per_tile + j * CHUNK
            pltpu.sync_copy(acc.at[pl.ds(r0, CHUNK)], zbuf)
            pltpu.sync_copy(zbuf, sum_hbm.at[c, pl.ds(r0, CHUNK)])
            if with_counts:
                pltpu.sync_copy(cacc.at[pl.ds(r0, CHUNK)], ones_v)
                pltpu.sync_copy(ones_v, cnt_hbm.at[c, pl.ds(r0, CHUNK)])
            return 0
        lax.fori_loop(0, nread, rd, 0)

    f = pl.kernel(body, out_type=tuple(outs), mesh=mesh,
                  scratch_types=tuple(scratch),
                  compiler_params=pltpu.CompilerParams(
                      use_tc_tiling_on_sc=False))
    return f(table, src3, dst3)


def _dot_nt(a, b):
    return lax.dot_general(a, b, (((1,), (1,)), ((), ())),
                           preferred_element_type=jnp.float32)


def _mean_block(sums_ref, cnt_ref, split_feat):
    cc = jnp.maximum(cnt_ref[0, :, 0:1] + cnt_ref[1, :, 0:1], 1.0)
    if split_feat:
        summed = jnp.concatenate([sums_ref[0], sums_ref[1]], axis=1)
    else:
        summed = sums_ref[0] + sums_ref[1]
    return summed / cc


def _dense_layer(sums, cnt, x, wl, bl, wr, g, b, n, split_feat):
    nrb = n // RB
    dh = sums.shape[2]
    dout = wl.shape[0]
    din = x.shape[1]

    def body(sums_ref, cnt_ref, x_ref, wl_ref, bl_ref, wr_ref, g_ref, b_ref,
             out_ref, t_s, st_s):
        p = pl.program_id(0)
        i = pl.program_id(1)

        @pl.when(p == 0)
        def _():
            mean = _mean_block(sums_ref, cnt_ref, split_feat)
            t = _dot_nt(mean, wl_ref[...]) + bl_ref[...] + \
                _dot_nt(x_ref[...], wr_ref[...])
            t_s[pl.ds(i * RB, RB), :] = t

            @pl.when(i == 0)
            def _():
                st_s[...] = jnp.zeros((2, dout), jnp.float32)
            st_s[0:1, :] += jnp.sum(t, axis=0, keepdims=True)
            st_s[1:2, :] += jnp.sum(t * t, axis=0, keepdims=True)

        @pl.when(p == 1)
        def _():
            t = t_s[pl.ds(i * RB, RB), :]
            mu = st_s[0:1, :] * (1.0 / n)
            var = st_s[1:2, :] * (1.0 / n) - mu * mu
            y = (t - mu) * lax.rsqrt(var + 1e-5) * g_ref[...] + b_ref[...]
            out_ref[...] = jnp.maximum(y, 0.0)

    return pl.pallas_call(
        body,
        grid=(2, nrb),
        in_specs=[
            pl.BlockSpec((NC, RB, dh), lambda p, i: (0, i, 0)),
            pl.BlockSpec((NC, RB, LN), lambda p, i: (0, i, 0)),
            pl.BlockSpec((RB, din), lambda p, i: (i, 0)),
            pl.BlockSpec((dout, din), lambda p, i: (0, 0)),
            pl.BlockSpec((1, dout), lambda p, i: (0, 0)),
            pl.BlockSpec((dout, din), lambda p, i: (0, 0)),
            pl.BlockSpec((1, dout), lambda p, i: (0, 0)),
            pl.BlockSpec((1, dout), lambda p, i: (0, 0)),
        ],
        out_specs=pl.BlockSpec((RB, dout), lambda p, i: (i, 0)),
        out_shape=jax.ShapeDtypeStruct((n, dout), jnp.float32),
        scratch_shapes=[
            pltpu.VMEM((n, dout), jnp.float32),
            pltpu.VMEM((2, dout), jnp.float32),
        ],
    )(sums, cnt, x, wl, bl, wr, g, b)


def _dense_layer_pool(sums, cnt, x, batch2, wl, bl, wr, g, b, n, ng,
                      split_feat):
    nrb = n // RB
    dh = sums.shape[2]
    dout = wl.shape[0]
    din = x.shape[1]

    def body(sums_ref, cnt_ref, x_ref, batch_ref, wl_ref, bl_ref, wr_ref,
             g_ref, b_ref, out_ref, t_s, st_s, ps_s, pm_s, pc_s):
        p = pl.program_id(0)
        i = pl.program_id(1)

        @pl.when(p == 0)
        def _():
            mean = _mean_block(sums_ref, cnt_ref, split_feat)
            t = _dot_nt(mean, wl_ref[...]) + bl_ref[...] + \
                _dot_nt(x_ref[...], wr_ref[...])
            t_s[pl.ds(i * RB, RB), :] = t

            @pl.when(i == 0)
            def _():
                st_s[...] = jnp.zeros((2, dout), jnp.float32)
            st_s[0:1, :] += jnp.sum(t, axis=0, keepdims=True)
            st_s[1:2, :] += jnp.sum(t * t, axis=0, keepdims=True)

        @pl.when(p == 1)
        def _():
            t = t_s[pl.ds(i * RB, RB), :]
            mu = st_s[0:1, :] * (1.0 / n)
            var = st_s[1:2, :] * (1.0 / n) - mu * mu
            f = jnp.maximum(
                (t - mu) * lax.rsqrt(var + 1e-5) * g_ref[...] + b_ref[...],
                0.0)

            @pl.when(i == 0)
            def _():
                ps_s[...] = jnp.zeros((ng, dout), jnp.float32)
                pm_s[...] = jnp.full((ng, dout), -jnp.inf, jnp.float32)
                pc_s[...] = jnp.zeros((ng, 1), jnp.float32)

            bb = batch_ref[...]
            lo = batch_ref[0, 0]
            hi = batch_ref[RB - 1, 0]

            def gpool(gi, _):
                mk = bb == gi
                fs = jnp.sum(jnp.where(mk, f, 0.0), axis=0, keepdims=True)
                fm = jnp.max(jnp.where(mk, f, -jnp.inf), axis=0,
                             keepdims=True)
                cs = jnp.sum(mk.astype(jnp.float32), axis=0, keepdims=True)
                ps_s[pl.ds(gi, 1), :] += fs
                pm_s[pl.ds(gi, 1), :] = jnp.maximum(pm_s[pl.ds(gi, 1), :], fm)
                pc_s[pl.ds(gi, 1), :] += cs
                return 0
            lax.fori_loop(lo, hi + 1, gpool, 0)

            @pl.when(i == nrb - 1)
            def _():
                cg = jnp.maximum(pc_s[...], 1.0)
                out_ref[...] = jnp.concatenate(
                    [ps_s[...] / cg, pm_s[...]], axis=1)

    return pl.pallas_call(
        body,
        grid=(2, nrb),
        in_specs=[
            pl.BlockSpec((NC, RB, dh), lambda p, i: (0, i, 0)),
            pl.BlockSpec((NC, RB, LN), lambda p, i: (0, i, 0)),
            pl.BlockSpec((RB, din), lambda p, i: (i, 0)),
            pl.BlockSpec((RB, 1), lambda p, i: (i, 0)),
            pl.BlockSpec((dout, din), lambda p, i: (0, 0)),
            pl.BlockSpec((1, dout), lambda p, i: (0, 0)),
            pl.BlockSpec((dout, din), lambda p, i: (0, 0)),
            pl.BlockSpec((1, dout), lambda p, i: (0, 0)),
            pl.BlockSpec((1, dout), lambda p, i: (0, 0)),
        ],
        out_specs=pl.BlockSpec((ng, 2 * dout), lambda p, i: (0, 0)),
        out_shape=jax.ShapeDtypeStruct((ng, 2 * dout), jnp.float32),
        scratch_shapes=[
            pltpu.VMEM((n, dout), jnp.float32),
            pltpu.VMEM((2, dout), jnp.float32),
            pltpu.VMEM((ng, dout), jnp.float32),
            pltpu.VMEM((ng, dout), jnp.float32),
            pltpu.VMEM((ng, 1), jnp.float32),
        ],
    )(sums, cnt, x, batch2, wl, bl, wr, g, b)


def _head(f0, f1, le0, le1, wf, bf, wc1, bc1, wc2, bc2):
    ng, dg = f0.shape
    nl = le0.shape[1]

    def body(f0_ref, f1_ref, le0_ref, le1_ref, wf_ref, bf_ref, wc1_ref,
             bc1_ref, wc2_ref, bc2_ref, out_ref):
        cat0 = jnp.concatenate([f0_ref[...], le0_ref[...]], axis=1)
        cat1 = jnp.concatenate([f1_ref[...], le1_ref[...]], axis=1)
        fused0 = _dot_nt(cat0, wf_ref[...]) + bf_ref[...]
        fused1 = _dot_nt(cat1, wf_ref[...]) + bf_ref[...]
        comb = jnp.concatenate(
            [fused0, fused1, jnp.abs(fused0 - fused1), fused0 * fused1],
            axis=1)
        h = jnp.maximum(_dot_nt(comb, wc1_ref[...]) + bc1_ref[...], 0.0)
        z = _dot_nt(h, wc2_ref[...])[:, 0:1] + bc2_ref[0, 0]
        out_ref[...] = 1.0 / (1.0 + jnp.exp(-z))

    return pl.pallas_call(
        body,
        out_shape=jax.ShapeDtypeStruct((ng, 1), jnp.float32),
    )(f0, f1, le0, le1, wf, bf, wc1, bc1, wc2, bc2)


def kernel(x0, edge_index0, batch0, logic0_embed, x1, edge_index1, batch1,
           logic1_embed, Wl1, bl1, Wr1, Wl2, bl2, Wr2, g1, b1, g2, b2,
           Wf, bf, Wc1, bc1, Wc2, bc2):
    n, din = x0.shape
    dh = Wl1.shape[0]
    ng = logic0_embed.shape[0]

    bl1r = bl1.reshape(1, -1)
    g1r = g1.reshape(1, -1)
    b1r = b1.reshape(1, -1)
    bl2r = bl2.reshape(1, -1)
    g2r = g2.reshape(1, -1)
    b2r = b2.reshape(1, -1)
    bfr = bf.reshape(1, -1)
    bc1r = bc1.reshape(1, -1)
    bc2r = bc2.reshape(1, -1)
    wc2r = jnp.tile(Wc2, (8, 1))

    def side(x, ei, batch):
        e = ei.shape[1]
        unit = NC * NS * CHUNK
        ep = -(-e // unit) * unit
        src = ei[0]
        dst = ei[1]
        if ep > e:
            pad = ep - e
            src = jnp.concatenate([src, jnp.zeros((pad,), jnp.int32)])
            dst = jnp.concatenate(
                [dst, jnp.full((pad,), NPAD - 8, jnp.int32)])
        sums1, cnt = _seg_sum_sc(x, src, dst, False, True)
        h1 = _dense_layer(sums1, cnt, x, Wl1, bl1r, Wr1, g1r, b1r, n, False)
        sums2 = _seg_sum_sc(
            h1.reshape(2 * n, dh // 2), src, dst, True, False)[0]
        pooled = _dense_layer_pool(
            sums2, cnt, h1, batch.reshape(n, 1), Wl2, bl2r, Wr2, g2r, b2r,
            n, ng, True)
        return pooled

    f0 = side(x0, edge_index0, batch0)
    f1 = side(x1, edge_index1, batch1)
    prob = _head(f0, f1, logic0_embed, logic1_embed, Wf, bfr, Wc1, bc1r,
                 wc2r, bc2r)
    return prob[:, 0]
